# baseline (speedup 1.0000x reference)
import functools

import jax
import jax.numpy as jnp
from jax import lax
from jax.experimental import pallas as pl
from jax.experimental.pallas import tpu as pltpu
from jax.experimental.pallas import tpu_sc as plsc

_THR = 0.6
_GAMMA = 0.9
_N_POS = 2048
_N_NEG = 8192
_N_TOT = _N_POS + _N_NEG
_CHUNK = 1024
_AP_STEPS = _N_POS // _CHUNK
_AN_STEPS = _N_NEG // _CHUNK
_A_STEPS = _AP_STEPS + _AN_STEPS
_B_STEPS = _N_POS // _CHUNK
_N_STEPS1 = _A_STEPS + _B_STEPS


@functools.cache
def _make_sc_gather():
    info = plsc.get_sparse_core_info()
    nc, ns = info.num_cores, info.num_subcores
    b_per_w = _N_POS // (nc * ns)
    mesh = plsc.VectorSubcoreMesh(core_axis_name="c", subcore_axis_name="s")

    @functools.partial(
        pl.kernel,
        out_type=(
            jax.ShapeDtypeStruct((1, _N_POS), jnp.float32),
            jax.ShapeDtypeStruct((1, _N_POS), jnp.float32),
        ),
        mesh=mesh,
        scratch_types=[
            pltpu.VMEM((b_per_w,), jnp.int32),
            pltpu.VMEM((b_per_w,), jnp.float32),
            pltpu.VMEM((b_per_w,), jnp.float32),
            pltpu.SemaphoreType.DMA,
        ],
    )
    def sc_gather(idx_hbm, u_all_hbm, u_pos_hbm, ua_out, up_out,
                  idx_v, a_v, p_v, sem):
        wid = lax.axis_index("s") * nc + lax.axis_index("c")
        base = wid * b_per_w
        pltpu.sync_copy(idx_hbm.at[pl.ds(base, b_per_w)], idx_v)
        pltpu.async_copy(u_all_hbm.at[idx_v], a_v, sem).wait()
        pltpu.async_copy(u_pos_hbm.at[idx_v], p_v, sem).wait()
        pltpu.sync_copy(a_v, ua_out.at[0, pl.ds(base, b_per_w)])
        pltpu.sync_copy(p_v, up_out.at[0, pl.ds(base, b_per_w)])

    return sc_gather


def _gather_u(index_s, u_all, u_pos):
    return _make_sc_gather()(index_s, u_all.reshape(-1), u_pos.reshape(-1))


def _sums_body(fp_col_ref, fn_col_ref, f_row_ref, idx_col_ref, idx_row_ref,
               s_out, p_out, w_out):
    i = pl.program_id(0)
    g_row = _THR - f_row_ref[...]

    @pl.when(i < _AP_STEPS)
    def _phase_a_pos():
        b = jnp.maximum(g_row + fp_col_ref[...], 0.0)
        part = jnp.sum(b * b, axis=0, keepdims=True)

        @pl.when(i == 0)
        def _():
            s_out[...] = jnp.zeros_like(s_out)
            p_out[...] = jnp.zeros_like(p_out)

        s_out[...] += part
        p_out[...] += part

    @pl.when(jnp.logical_and(i >= _AP_STEPS, i < _A_STEPS))
    def _phase_a_neg():
        b = jnp.maximum(g_row + fn_col_ref[...], 0.0)
        s_out[...] += jnp.sum(b * b, axis=0, keepdims=True)

    @pl.when(i >= _A_STEPS)
    def _phase_b():
        eq = idx_col_ref[...] == idx_row_ref[...]
        kk = (lax.broadcasted_iota(jnp.int32, eq.shape, 0)
              + (i - _A_STEPS) * _CHUNK)
        part = jnp.max(jnp.where(eq, kk, -1), axis=0, keepdims=True)

        @pl.when(i == _A_STEPS)
        def _():
            w_out[...] = part

        @pl.when(i > _A_STEPS)
        def _():
            w_out[...] = jnp.maximum(w_out[...], part)


def _sums(f_ps, f_ns, index_s):
    full = lambda i: (0, 0)
    return pl.pallas_call(
        _sums_body,
        grid=(_N_STEPS1,),
        in_specs=[
            pl.BlockSpec((_CHUNK, 1),
                         lambda i: (jnp.minimum(i, _AP_STEPS - 1), 0)),
            pl.BlockSpec((_CHUNK, 1),
                         lambda i: (jnp.clip(i - _AP_STEPS, 0, _AN_STEPS - 1), 0)),
            pl.BlockSpec((1, _N_POS), full),
            pl.BlockSpec((_CHUNK, 1),
                         lambda i: (jnp.clip(i - _A_STEPS, 0, _B_STEPS - 1), 0)),
            pl.BlockSpec((1, _N_POS), full),
        ],
        out_specs=[
            pl.BlockSpec((1, _N_POS), full),
            pl.BlockSpec((1, _N_POS), full),
            pl.BlockSpec((1, _N_POS), full),
        ],
        out_shape=[
            jax.ShapeDtypeStruct((1, _N_POS), jnp.float32),
            jax.ShapeDtypeStruct((1, _N_POS), jnp.float32),
            jax.ShapeDtypeStruct((1, _N_POS), jnp.int32),
        ],
    )(f_ps.reshape(_N_POS, 1), f_ns.reshape(_N_NEG, 1),
      f_ps.reshape(1, _N_POS), index_s.reshape(_N_POS, 1),
      index_s.reshape(1, _N_POS))


def _combine_body(s_ref, p_ref, w_ref, ua0_ref, up0_ref, out_ref):
    w = w_ref[...]
    sw = jnp.zeros((1, _N_POS), jnp.float32)
    pw = jnp.zeros((1, _N_POS), jnp.float32)
    for t in range(_B_STEPS):
        jj = lax.broadcasted_iota(jnp.int32, (_CHUNK, _N_POS), 0) + t * _CHUNK
        ind = (jj == w).astype(jnp.float32)
        sl = slice(t * _CHUNK, (t + 1) * _CHUNK)
        sw += jnp.dot(s_ref[0:1, sl], ind,
                      preferred_element_type=jnp.float32,
                      precision=lax.Precision.HIGHEST)
        pw += jnp.dot(p_ref[0:1, sl], ind,
                      preferred_element_type=jnp.float32,
                      precision=lax.Precision.HIGHEST)
    inv_n = 1.0 / _N_TOT
    ua = (1.0 - _GAMMA) * ua0_ref[...] + _GAMMA * (sw * inv_n)
    up = (1.0 - _GAMMA) * up0_ref[...] + _GAMMA * (pw * inv_n)
    term = (up * s_ref[...] - ua * p_ref[...]) / (ua * ua)
    out_ref[...] = jnp.sum(term, axis=1, keepdims=True) * (
        1.0 / (_N_POS * _N_TOT))


def _combine(s, p, w, ua0, up0):
    return pl.pallas_call(
        _combine_body,
        out_shape=jax.ShapeDtypeStruct((1, 1), jnp.float32),
    )(s, p, w, ua0, up0)


def kernel(f_ps, f_ns, index_s, u_all, u_pos):
    ua0, up0 = _gather_u(index_s, u_all, u_pos)
    s, p, w = _sums(f_ps, f_ns, index_s)
    out = _combine(s, p, w, ua0, up0)
    return out[0, 0]


# speedup vs baseline: 1.0259x; 1.0259x over previous
import functools

import jax
import jax.numpy as jnp
from jax import lax
from jax.experimental import pallas as pl
from jax.experimental.pallas import tpu as pltpu
from jax.experimental.pallas import tpu_sc as plsc

_THR = 0.6
_GAMMA = 0.9
_N_POS = 2048
_N_NEG = 8192
_N_TOT = _N_POS + _N_NEG
_CHUNK = 1024
_AP_STEPS = _N_POS // _CHUNK
_AN_STEPS = _N_NEG // _CHUNK
_A_STEPS = _AP_STEPS + _AN_STEPS
_B_STEPS = _N_POS // _CHUNK
_N_STEPS1 = _A_STEPS + _B_STEPS


@functools.cache
def _make_sc_gather():
    info = plsc.get_sparse_core_info()
    nc, ns = 1, info.num_subcores
    b_per_w = _N_POS // (nc * ns)
    mesh = plsc.VectorSubcoreMesh(core_axis_name="c", subcore_axis_name="s",
                                  num_cores=nc)

    @functools.partial(
        pl.kernel,
        out_type=(
            jax.ShapeDtypeStruct((1, _N_POS), jnp.float32),
            jax.ShapeDtypeStruct((1, _N_POS), jnp.float32),
        ),
        mesh=mesh,
        scratch_types=[
            pltpu.VMEM((b_per_w,), jnp.int32),
            pltpu.VMEM((b_per_w,), jnp.float32),
            pltpu.VMEM((b_per_w,), jnp.float32),
            pltpu.SemaphoreType.DMA,
        ],
    )
    def sc_gather(idx_hbm, u_all_hbm, u_pos_hbm, ua_out, up_out,
                  idx_v, a_v, p_v, sem):
        wid = lax.axis_index("s") * nc + lax.axis_index("c")
        base = wid * b_per_w
        pltpu.sync_copy(idx_hbm.at[pl.ds(base, b_per_w)], idx_v)
        pltpu.async_copy(u_all_hbm.at[idx_v], a_v, sem).wait()
        pltpu.async_copy(u_pos_hbm.at[idx_v], p_v, sem).wait()
        pltpu.sync_copy(a_v, ua_out.at[0, pl.ds(base, b_per_w)])
        pltpu.sync_copy(p_v, up_out.at[0, pl.ds(base, b_per_w)])

    return sc_gather


def _gather_u(index_s, u_all, u_pos):
    return _make_sc_gather()(index_s, u_all.reshape(-1), u_pos.reshape(-1))


def _sums_body(fp_col_ref, fn_col_ref, f_row_ref, idx_col_ref, idx_row_ref,
               s_out, p_out, w_out):
    i = pl.program_id(0)
    g_row = _THR - f_row_ref[...]

    @pl.when(i < _AP_STEPS)
    def _phase_a_pos():
        b = jnp.maximum(g_row + fp_col_ref[...], 0.0)
        part = jnp.sum(b * b, axis=0, keepdims=True)

        @pl.when(i == 0)
        def _():
            s_out[...] = jnp.zeros_like(s_out)
            p_out[...] = jnp.zeros_like(p_out)

        s_out[...] += part
        p_out[...] += part

    @pl.when(jnp.logical_and(i >= _AP_STEPS, i < _A_STEPS))
    def _phase_a_neg():
        b = jnp.maximum(g_row + fn_col_ref[...], 0.0)
        s_out[...] += jnp.sum(b * b, axis=0, keepdims=True)

    @pl.when(i >= _A_STEPS)
    def _phase_b():
        eq = idx_col_ref[...] == idx_row_ref[...]
        kk = (lax.broadcasted_iota(jnp.int32, eq.shape, 0)
              + (i - _A_STEPS) * _CHUNK)
        part = jnp.max(jnp.where(eq, kk, -1), axis=0, keepdims=True)

        @pl.when(i == _A_STEPS)
        def _():
            w_out[...] = part

        @pl.when(i > _A_STEPS)
        def _():
            w_out[...] = jnp.maximum(w_out[...], part)


def _sums(f_ps, f_ns, index_s):
    full = lambda i: (0, 0)
    return pl.pallas_call(
        _sums_body,
        grid=(_N_STEPS1,),
        in_specs=[
            pl.BlockSpec((_CHUNK, 1),
                         lambda i: (jnp.minimum(i, _AP_STEPS - 1), 0)),
            pl.BlockSpec((_CHUNK, 1),
                         lambda i: (jnp.clip(i - _AP_STEPS, 0, _AN_STEPS - 1), 0)),
            pl.BlockSpec((1, _N_POS), full),
            pl.BlockSpec((_CHUNK, 1),
                         lambda i: (jnp.clip(i - _A_STEPS, 0, _B_STEPS - 1), 0)),
            pl.BlockSpec((1, _N_POS), full),
        ],
        out_specs=[
            pl.BlockSpec((1, _N_POS), full),
            pl.BlockSpec((1, _N_POS), full),
            pl.BlockSpec((1, _N_POS), full),
        ],
        out_shape=[
            jax.ShapeDtypeStruct((1, _N_POS), jnp.float32),
            jax.ShapeDtypeStruct((1, _N_POS), jnp.float32),
            jax.ShapeDtypeStruct((1, _N_POS), jnp.int32),
        ],
    )(f_ps.reshape(_N_POS, 1), f_ns.reshape(_N_NEG, 1),
      f_ps.reshape(1, _N_POS), index_s.reshape(_N_POS, 1),
      index_s.reshape(1, _N_POS))


def _combine_body(s_ref, p_ref, w_ref, ua0_ref, up0_ref, out_ref):
    w = w_ref[...]
    sw = jnp.zeros((1, _N_POS), jnp.float32)
    pw = jnp.zeros((1, _N_POS), jnp.float32)
    for t in range(_B_STEPS):
        jj = lax.broadcasted_iota(jnp.int32, (_CHUNK, _N_POS), 0) + t * _CHUNK
        ind = (jj == w).astype(jnp.float32)
        sl = slice(t * _CHUNK, (t + 1) * _CHUNK)
        sw += jnp.dot(s_ref[0:1, sl], ind,
                      preferred_element_type=jnp.float32,
                      precision=lax.Precision.HIGHEST)
        pw += jnp.dot(p_ref[0:1, sl], ind,
                      preferred_element_type=jnp.float32,
                      precision=lax.Precision.HIGHEST)
    inv_n = 1.0 / _N_TOT
    ua = (1.0 - _GAMMA) * ua0_ref[...] + _GAMMA * (sw * inv_n)
    up = (1.0 - _GAMMA) * up0_ref[...] + _GAMMA * (pw * inv_n)
    term = (up * s_ref[...] - ua * p_ref[...]) / (ua * ua)
    out_ref[...] = jnp.sum(term, axis=1, keepdims=True) * (
        1.0 / (_N_POS * _N_TOT))


def _combine(s, p, w, ua0, up0):
    return pl.pallas_call(
        _combine_body,
        out_shape=jax.ShapeDtypeStruct((1, 1), jnp.float32),
    )(s, p, w, ua0, up0)


def kernel(f_ps, f_ns, index_s, u_all, u_pos):
    ua0, up0 = _gather_u(index_s, u_all, u_pos)
    s, p, w = _sums(f_ps, f_ns, index_s)
    out = _combine(s, p, w, ua0, up0)
    return out[0, 0]


# speedup vs baseline: 1.1216x; 1.0933x over previous
import functools

import jax
import jax.numpy as jnp
from jax import lax
from jax.experimental import pallas as pl
from jax.experimental.pallas import tpu as pltpu
from jax.experimental.pallas import tpu_sc as plsc

_THR = 0.6
_GAMMA = 0.9
_N_POS = 2048
_N_NEG = 8192
_N_TOT = _N_POS + _N_NEG
_CHUNK = 1024
_AP_STEPS = _N_POS // _CHUNK
_AN_STEPS = _N_NEG // _CHUNK
_A_STEPS = _AP_STEPS + _AN_STEPS
_B_STEPS = _N_POS // _CHUNK
_N_STEPS1 = _A_STEPS + _B_STEPS


@functools.cache
def _make_sc_gather():
    info = plsc.get_sparse_core_info()
    nc, ns = info.num_cores, info.num_subcores
    b_per_w = _N_POS // (nc * ns)
    mesh = plsc.VectorSubcoreMesh(core_axis_name="c", subcore_axis_name="s")

    @functools.partial(
        pl.kernel,
        out_type=(
            jax.ShapeDtypeStruct((1, _N_POS), jnp.float32),
            jax.ShapeDtypeStruct((1, _N_POS), jnp.float32),
        ),
        mesh=mesh,
        scratch_types=[
            pltpu.VMEM((b_per_w,), jnp.int32),
            pltpu.VMEM((b_per_w,), jnp.float32),
            pltpu.VMEM((b_per_w,), jnp.float32),
            pltpu.SemaphoreType.DMA,
        ],
    )
    def sc_gather(idx_hbm, u_all_hbm, u_pos_hbm, ua_out, up_out,
                  idx_v, a_v, p_v, sem):
        wid = lax.axis_index("s") * nc + lax.axis_index("c")
        base = wid * b_per_w
        pltpu.sync_copy(idx_hbm.at[pl.ds(base, b_per_w)], idx_v)
        pltpu.async_copy(u_all_hbm.at[idx_v], a_v, sem).wait()
        pltpu.async_copy(u_pos_hbm.at[idx_v], p_v, sem).wait()
        pltpu.sync_copy(a_v, ua_out.at[0, pl.ds(base, b_per_w)])
        pltpu.sync_copy(p_v, up_out.at[0, pl.ds(base, b_per_w)])

    return sc_gather


def _gather_u(index_s, u_all, u_pos):
    return _make_sc_gather()(index_s, u_all.reshape(-1), u_pos.reshape(-1))


def _sums_body(fp_col_ref, fn_col_ref, f_row_ref, idx_col_ref, idx_row_ref,
               s_out, p_out, w_out):
    i = pl.program_id(0)
    g_row = _THR - f_row_ref[...]

    @pl.when(i < _AP_STEPS)
    def _phase_a_pos():
        b = jnp.maximum(g_row + fp_col_ref[...], 0.0)
        part = jnp.sum(b * b, axis=0, keepdims=True)

        @pl.when(i == 0)
        def _():
            s_out[...] = jnp.zeros_like(s_out)
            p_out[...] = jnp.zeros_like(p_out)

        s_out[...] += part
        p_out[...] += part

    @pl.when(jnp.logical_and(i >= _AP_STEPS, i < _A_STEPS))
    def _phase_a_neg():
        b = jnp.maximum(g_row + fn_col_ref[...], 0.0)
        s_out[...] += jnp.sum(b * b, axis=0, keepdims=True)

    @pl.when(i >= _A_STEPS)
    def _phase_b():
        eq = idx_col_ref[...] == idx_row_ref[...]
        kk = (lax.broadcasted_iota(jnp.int32, eq.shape, 0)
              + (i - _A_STEPS) * _CHUNK)
        part = jnp.max(jnp.where(eq, kk, -1), axis=0, keepdims=True)

        @pl.when(i == _A_STEPS)
        def _():
            w_out[...] = part

        @pl.when(i > _A_STEPS)
        def _():
            w_out[...] = jnp.maximum(w_out[...], part)


def _sums(f_ps, f_ns, index_s):
    full = lambda i: (0, 0)
    return pl.pallas_call(
        _sums_body,
        grid=(_N_STEPS1,),
        in_specs=[
            pl.BlockSpec((_CHUNK, 1),
                         lambda i: (jnp.minimum(i, _AP_STEPS - 1), 0)),
            pl.BlockSpec((_CHUNK, 1),
                         lambda i: (jnp.clip(i - _AP_STEPS, 0, _AN_STEPS - 1), 0)),
            pl.BlockSpec((1, _N_POS), full),
            pl.BlockSpec((_CHUNK, 1),
                         lambda i: (jnp.clip(i - _A_STEPS, 0, _B_STEPS - 1), 0)),
            pl.BlockSpec((1, _N_POS), full),
        ],
        out_specs=[
            pl.BlockSpec((1, _N_POS), full),
            pl.BlockSpec((1, _N_POS), full),
            pl.BlockSpec((1, _N_POS), full),
        ],
        out_shape=[
            jax.ShapeDtypeStruct((1, _N_POS), jnp.float32),
            jax.ShapeDtypeStruct((1, _N_POS), jnp.float32),
            jax.ShapeDtypeStruct((1, _N_POS), jnp.int32),
        ],
    )(f_ps.reshape(_N_POS, 1), f_ns.reshape(_N_NEG, 1),
      f_ps.reshape(1, _N_POS), index_s.reshape(_N_POS, 1),
      index_s.reshape(1, _N_POS))


def _combine_body(s_ref, p_ref, s_col_ref, p_col_ref, w_ref, ua0_ref,
                  up0_ref, out_ref):
    w = w_ref[...]
    sw = jnp.zeros((1, _N_POS), jnp.float32)
    pw = jnp.zeros((1, _N_POS), jnp.float32)
    for t in range(_B_STEPS):
        jj = lax.broadcasted_iota(jnp.int32, (_CHUNK, _N_POS), 0) + t * _CHUNK
        ind = jj == w
        sl = pl.ds(t * _CHUNK, _CHUNK)
        sw += jnp.sum(jnp.where(ind, s_col_ref[sl, 0:1], 0.0),
                      axis=0, keepdims=True)
        pw += jnp.sum(jnp.where(ind, p_col_ref[sl, 0:1], 0.0),
                      axis=0, keepdims=True)
    inv_n = 1.0 / _N_TOT
    ua = (1.0 - _GAMMA) * ua0_ref[...] + _GAMMA * (sw * inv_n)
    up = (1.0 - _GAMMA) * up0_ref[...] + _GAMMA * (pw * inv_n)
    term = (up * s_ref[...] - ua * p_ref[...]) / (ua * ua)
    out_ref[...] = jnp.sum(term, axis=1, keepdims=True) * (
        1.0 / (_N_POS * _N_TOT))


def _combine(s, p, w, ua0, up0):
    return pl.pallas_call(
        _combine_body,
        out_shape=jax.ShapeDtypeStruct((1, 1), jnp.float32),
    )(s, p, s.reshape(_N_POS, 1), p.reshape(_N_POS, 1), w, ua0, up0)


def kernel(f_ps, f_ns, index_s, u_all, u_pos):
    ua0, up0 = _gather_u(index_s, u_all, u_pos)
    s, p, w = _sums(f_ps, f_ns, index_s)
    out = _combine(s, p, w, ua0, up0)
    return out[0, 0]


# speedup vs baseline: 1.1382x; 1.0148x over previous
import functools

import jax
import jax.numpy as jnp
from jax import lax
from jax.experimental import pallas as pl
from jax.experimental.pallas import tpu as pltpu
from jax.experimental.pallas import tpu_sc as plsc

_THR = 0.6
_GAMMA = 0.9
_N_POS = 2048
_N_NEG = 8192
_N_TOT = _N_POS + _N_NEG
_CHUNK = 2048
_AP_STEPS = _N_POS // _CHUNK
_AN_STEPS = _N_NEG // _CHUNK
_A_STEPS = _AP_STEPS + _AN_STEPS
_B_STEPS = _N_POS // _CHUNK
_N_STEPS1 = _A_STEPS + _B_STEPS


@functools.cache
def _make_sc_gather():
    info = plsc.get_sparse_core_info()
    nc, ns = info.num_cores, info.num_subcores
    b_per_w = _N_POS // (nc * ns)
    mesh = plsc.VectorSubcoreMesh(core_axis_name="c", subcore_axis_name="s")

    @functools.partial(
        pl.kernel,
        out_type=(
            jax.ShapeDtypeStruct((1, _N_POS), jnp.float32),
            jax.ShapeDtypeStruct((1, _N_POS), jnp.float32),
        ),
        mesh=mesh,
        scratch_types=[
            pltpu.VMEM((b_per_w,), jnp.int32),
            pltpu.VMEM((b_per_w,), jnp.float32),
            pltpu.VMEM((b_per_w,), jnp.float32),
            pltpu.SemaphoreType.DMA,
        ],
    )
    def sc_gather(idx_hbm, u_all_hbm, u_pos_hbm, ua_out, up_out,
                  idx_v, a_v, p_v, sem):
        wid = lax.axis_index("s") * nc + lax.axis_index("c")
        base = wid * b_per_w
        pltpu.sync_copy(idx_hbm.at[pl.ds(base, b_per_w)], idx_v)
        pltpu.async_copy(u_all_hbm.at[idx_v], a_v, sem).wait()
        pltpu.async_copy(u_pos_hbm.at[idx_v], p_v, sem).wait()
        pltpu.sync_copy(a_v, ua_out.at[0, pl.ds(base, b_per_w)])
        pltpu.sync_copy(p_v, up_out.at[0, pl.ds(base, b_per_w)])

    return sc_gather


def _gather_u(index_s, u_all, u_pos):
    return _make_sc_gather()(index_s, u_all.reshape(-1), u_pos.reshape(-1))


def _sums_body(fp_col_ref, fn_col_ref, f_row_ref, idx_col_ref, idx_row_ref,
               s_out, p_out, w_out):
    i = pl.program_id(0)
    g_row = _THR - f_row_ref[...]

    @pl.when(i < _AP_STEPS)
    def _phase_a_pos():
        b = jnp.maximum(g_row + fp_col_ref[...], 0.0)
        part = jnp.sum(b * b, axis=0, keepdims=True)

        @pl.when(i == 0)
        def _():
            s_out[...] = jnp.zeros_like(s_out)
            p_out[...] = jnp.zeros_like(p_out)

        s_out[...] += part
        p_out[...] += part

    @pl.when(jnp.logical_and(i >= _AP_STEPS, i < _A_STEPS))
    def _phase_a_neg():
        b = jnp.maximum(g_row + fn_col_ref[...], 0.0)
        s_out[...] += jnp.sum(b * b, axis=0, keepdims=True)

    @pl.when(i >= _A_STEPS)
    def _phase_b():
        eq = idx_col_ref[...] == idx_row_ref[...]
        kk = (lax.broadcasted_iota(jnp.int32, eq.shape, 0)
              + (i - _A_STEPS) * _CHUNK)
        part = jnp.max(jnp.where(eq, kk, -1), axis=0, keepdims=True)

        @pl.when(i == _A_STEPS)
        def _():
            w_out[...] = part

        @pl.when(i > _A_STEPS)
        def _():
            w_out[...] = jnp.maximum(w_out[...], part)


def _sums(f_ps, f_ns, index_s):
    full = lambda i: (0, 0)
    return pl.pallas_call(
        _sums_body,
        grid=(_N_STEPS1,),
        in_specs=[
            pl.BlockSpec((_CHUNK, 1),
                         lambda i: (jnp.minimum(i, _AP_STEPS - 1), 0)),
            pl.BlockSpec((_CHUNK, 1),
                         lambda i: (jnp.clip(i - _AP_STEPS, 0, _AN_STEPS - 1), 0)),
            pl.BlockSpec((1, _N_POS), full),
            pl.BlockSpec((_CHUNK, 1),
                         lambda i: (jnp.clip(i - _A_STEPS, 0, _B_STEPS - 1), 0)),
            pl.BlockSpec((1, _N_POS), full),
        ],
        out_specs=[
            pl.BlockSpec((1, _N_POS), full),
            pl.BlockSpec((1, _N_POS), full),
            pl.BlockSpec((1, _N_POS), full),
        ],
        out_shape=[
            jax.ShapeDtypeStruct((1, _N_POS), jnp.float32),
            jax.ShapeDtypeStruct((1, _N_POS), jnp.float32),
            jax.ShapeDtypeStruct((1, _N_POS), jnp.int32),
        ],
    )(f_ps.reshape(_N_POS, 1), f_ns.reshape(_N_NEG, 1),
      f_ps.reshape(1, _N_POS), index_s.reshape(_N_POS, 1),
      index_s.reshape(1, _N_POS))


def _combine_body(s_ref, p_ref, s_col_ref, p_col_ref, w_ref, ua0_ref,
                  up0_ref, out_ref):
    w = w_ref[...]
    sw = jnp.zeros((1, _N_POS), jnp.float32)
    pw = jnp.zeros((1, _N_POS), jnp.float32)
    for t in range(_B_STEPS):
        jj = lax.broadcasted_iota(jnp.int32, (_CHUNK, _N_POS), 0) + t * _CHUNK
        ind = jj == w
        sl = pl.ds(t * _CHUNK, _CHUNK)
        sw += jnp.sum(jnp.where(ind, s_col_ref[sl, 0:1], 0.0),
                      axis=0, keepdims=True)
        pw += jnp.sum(jnp.where(ind, p_col_ref[sl, 0:1], 0.0),
                      axis=0, keepdims=True)
    inv_n = 1.0 / _N_TOT
    ua = (1.0 - _GAMMA) * ua0_ref[...] + _GAMMA * (sw * inv_n)
    up = (1.0 - _GAMMA) * up0_ref[...] + _GAMMA * (pw * inv_n)
    term = (up * s_ref[...] - ua * p_ref[...]) / (ua * ua)
    out_ref[...] = jnp.sum(term, axis=1, keepdims=True) * (
        1.0 / (_N_POS * _N_TOT))


def _combine(s, p, w, ua0, up0):
    return pl.pallas_call(
        _combine_body,
        out_shape=jax.ShapeDtypeStruct((1, 1), jnp.float32),
    )(s, p, s.reshape(_N_POS, 1), p.reshape(_N_POS, 1), w, ua0, up0)


def kernel(f_ps, f_ns, index_s, u_all, u_pos):
    ua0, up0 = _gather_u(index_s, u_all, u_pos)
    s, p, w = _sums(f_ps, f_ns, index_s)
    out = _combine(s, p, w, ua0, up0)
    return out[0, 0]
